# baseline (speedup 1.0000x reference)
import functools

import jax
import jax.numpy as jnp
from jax import lax
from jax.experimental import pallas as pl
from jax.experimental.pallas import tpu as pltpu
from jax.experimental.pallas import tpu_sc as plsc

B = 16384
D = 64
H = 128
NC = 2
NS = 16
NW = NC * NS
BPW = B // NW
CHUNK = 128
NCH = BPW // CHUNK

BLK = 2048


@functools.lru_cache(maxsize=None)
def _gather_fn():
    mesh = plsc.VectorSubcoreMesh(core_axis_name="c", subcore_axis_name="s")

    @functools.partial(
        pl.kernel,
        mesh=mesh,
        compiler_params=pltpu.CompilerParams(use_tc_tiling_on_sc=False),
        out_type=[
            jax.ShapeDtypeStruct((B, D), jnp.float32),
            jax.ShapeDtypeStruct((B, D), jnp.float32),
        ],
        scratch_types=[
            pltpu.VMEM((NCH, CHUNK), jnp.int32),
            pltpu.VMEM((NCH, CHUNK), jnp.int32),
            pltpu.VMEM((BPW, D), jnp.float32),
            pltpu.VMEM((BPW, D), jnp.float32),
            pltpu.SemaphoreType.DMA,
            pltpu.SemaphoreType.DMA,
        ],
    )
    def gather(user_hbm, book_hbm, utab_hbm, btab_hbm, u_out, b_out,
               uidx, bidx, urows, brows, sem_u, sem_b):
        wid = lax.axis_index("s") * NC + lax.axis_index("c")
        base = wid * BPW
        pltpu.sync_copy(user_hbm.at[wid], uidx)
        pltpu.sync_copy(book_hbm.at[wid], bidx)
        copies = []
        for j in range(NCH):
            copies.append(pltpu.async_copy(
                utab_hbm.at[uidx.at[j]],
                urows.at[pl.ds(j * CHUNK, CHUNK)], sem_u))
            copies.append(pltpu.async_copy(
                btab_hbm.at[bidx.at[j]],
                brows.at[pl.ds(j * CHUNK, CHUNK)], sem_b))
        for c in copies:
            c.wait()
        pltpu.sync_copy(urows, u_out.at[pl.ds(base, BPW)])
        pltpu.sync_copy(brows, b_out.at[pl.ds(base, BPW)])

    return gather


def _combine_body(tag_ref, w_ref, b_ref, u_ref, bk_ref, out_ref):
    proj = jnp.dot(tag_ref[...], w_ref[...],
                   preferred_element_type=jnp.float32)
    integ = bk_ref[...] + proj + b_ref[...]
    out_ref[...] = jnp.sum(u_ref[...] * integ, axis=1)


def _combine(tag, W, b2, U, Bk):
    return pl.pallas_call(
        _combine_body,
        grid=(B // BLK,),
        in_specs=[
            pl.BlockSpec((BLK, H), lambda i: (i, 0)),
            pl.BlockSpec((H, D), lambda i: (0, 0)),
            pl.BlockSpec((1, D), lambda i: (0, 0)),
            pl.BlockSpec((BLK, D), lambda i: (i, 0)),
            pl.BlockSpec((BLK, D), lambda i: (i, 0)),
        ],
        out_specs=pl.BlockSpec((BLK,), lambda i: (i,)),
        out_shape=jax.ShapeDtypeStruct((B,), jnp.float32),
    )(tag, W, b2, U, Bk)


def kernel(user, book, tag_embedding, user_table, book_table, W_lin, b_lin):
    U, Bk = _gather_fn()(
        user.reshape(NW, NCH, CHUNK),
        book.reshape(NW, NCH, CHUNK),
        user_table, book_table)
    return _combine(tag_embedding, W_lin, b_lin.reshape(1, D), U, Bk)


# speedup vs baseline: 2.2807x; 2.2807x over previous
import functools

import jax
import jax.numpy as jnp
from jax import lax
from jax.experimental import pallas as pl
from jax.experimental.pallas import tpu as pltpu
from jax.experimental.pallas import tpu_sc as plsc

B = 16384
D = 64
H = 128
NC = 2
NS = 16
NW = NC * NS
BPW = B // NW
NG = BPW // 16

BLK = 2048


@functools.lru_cache(maxsize=None)
def _gather_fn():
    mesh = plsc.VectorSubcoreMesh(core_axis_name="c", subcore_axis_name="s")

    @functools.partial(
        pl.kernel,
        mesh=mesh,
        out_type=[
            jax.ShapeDtypeStruct((B // 8, 8, D), jnp.float32),
            jax.ShapeDtypeStruct((B // 8, 8, D), jnp.float32),
        ],
        scratch_types=[
            pltpu.VMEM((BPW,), jnp.int32),
            pltpu.VMEM((BPW,), jnp.int32),
            pltpu.VMEM((BPW // 8, 8, D), jnp.float32),
            pltpu.SemaphoreType.DMA,
        ],
    )
    def gather(user_hbm, book_hbm, utab3, btab3, u_out, b_out,
               uidx, bidx, buf, sem):
        wid = lax.axis_index("s") * NC + lax.axis_index("c")
        base = wid * BPW
        pltpu.sync_copy(user_hbm.at[pl.ds(base, BPW)], uidx)
        pltpu.sync_copy(book_hbm.at[pl.ds(base, BPW)], bidx)

        def process(idx_ref, tab3):
            def drain16():
                for _ in range(16):
                    pltpu.make_async_copy(
                        tab3.at[0, 0], buf.at[0, 0], sem).wait()

            def body(g, _):
                idxv = idx_ref[pl.ds(g * 16, 16)]
                tv = lax.shift_right_logical(idxv, 3)
                sv = lax.bitwise_and(idxv, 7)
                for lane in range(16):
                    row = g * 2 + lane // 8
                    pltpu.async_copy(
                        tab3.at[tv[lane], sv[lane]],
                        buf.at[row, lane % 8], sem)
                @pl.when(g > 0)
                def _d():
                    drain16()
                return _
            lax.fori_loop(0, NG, body, None)
            drain16()

        process(uidx, utab3)
        pltpu.sync_copy(buf, u_out.at[pl.ds(wid * (BPW // 8), BPW // 8)])
        process(bidx, btab3)
        pltpu.sync_copy(buf, b_out.at[pl.ds(wid * (BPW // 8), BPW // 8)])

    return gather


def _combine_body(tag_ref, w_ref, b_ref, u_ref, bk_ref, out_ref):
    proj = jnp.dot(tag_ref[...], w_ref[...],
                   preferred_element_type=jnp.float32)
    integ = bk_ref[...] + proj + b_ref[...]
    out_ref[...] = jnp.sum(u_ref[...] * integ, axis=1)


def _combine(tag, W, b2, U, Bk):
    return pl.pallas_call(
        _combine_body,
        grid=(B // BLK,),
        in_specs=[
            pl.BlockSpec((BLK, H), lambda i: (i, 0)),
            pl.BlockSpec((H, D), lambda i: (0, 0)),
            pl.BlockSpec((1, D), lambda i: (0, 0)),
            pl.BlockSpec((BLK, D), lambda i: (i, 0)),
            pl.BlockSpec((BLK, D), lambda i: (i, 0)),
        ],
        out_specs=pl.BlockSpec((BLK,), lambda i: (i,)),
        out_shape=jax.ShapeDtypeStruct((B,), jnp.float32),
    )(tag, W, b2, U, Bk)


def kernel(user, book, tag_embedding, user_table, book_table, W_lin, b_lin):
    U3, Bk3 = _gather_fn()(
        user, book,
        user_table.reshape(user_table.shape[0] // 8, 8, D),
        book_table.reshape(book_table.shape[0] // 8, 8, D))
    return _combine(tag_embedding, W_lin, b_lin.reshape(1, D),
                    U3.reshape(B, D), Bk3.reshape(B, D))
